# Initial kernel scaffold; baseline (speedup 1.0000x reference)
import jax
import jax.numpy as jnp
from jax.experimental import pallas as pl


def kernel(x, edge_index, W_l0, b_l0, W_r0, W_l1, b_l1, W_r1):
    raise NotImplementedError("write your pallas kernel here")


# speedup vs baseline: 4.7862x; 4.7862x over previous
import functools

import jax
import jax.numpy as jnp
from jax import lax
from jax.experimental import pallas as pl
from jax.experimental.pallas import tpu as pltpu
from jax.experimental.pallas import tpu_sc as plsc

N = 10000
E = 320000
D = 128
NC = 2
NS = 16
NW = NC * NS
CH = 80
EPW = E // NW
ROWS_PW = EPW // CH
NPT = N // NS


def _sc_agg_body(x_hbm, src_hbm, dst_hbm, acc_hbm, src_v, dst_v, rows_v, sem,
                 acc_sh):
    c = lax.axis_index("c")
    s = lax.axis_index("s")
    g = c * NS + s

    z16 = jnp.zeros((16,), jnp.float32)

    @pl.loop(0, CH)
    def _(r):
        for k in range(D // 16):
            rows_v[r, pl.ds(k * 16, 16)] = z16

    for r in range(NPT // CH):
        pltpu.sync_copy(rows_v, acc_sh.at[pl.ds(s * NPT + r * CH, CH)])
    rem = NPT % CH
    pltpu.sync_copy(rows_v.at[pl.ds(0, rem)],
                    acc_sh.at[pl.ds(s * NPT + NPT - rem, rem)])
    plsc.subcore_barrier()

    pltpu.sync_copy(src_hbm.at[g], src_v)
    pltpu.sync_copy(dst_hbm.at[g], dst_v)

    @pl.loop(0, ROWS_PW)
    def _(j):
        pltpu.async_copy(x_hbm.at[src_v.at[j]], rows_v, sem).wait()
        pltpu.sync_copy(rows_v, acc_sh.at[dst_v.at[j]], add=True)

    plsc.subcore_barrier()
    pltpu.sync_copy(acc_sh.at[pl.ds(s * NPT, NPT)], acc_hbm.at[c, s])


def _sc_deg_body(dst_hbm, deg_hbm, dst_v, deg_v):
    c = lax.axis_index("c")
    s = lax.axis_index("s")
    g = c * NS + s

    z16 = jnp.zeros((16,), jnp.float32)

    @pl.loop(0, N, step=16)
    def _(i):
        deg_v[pl.ds(i, 16)] = z16

    pltpu.sync_copy(dst_hbm.at[g], dst_v)

    ones16 = jnp.full((16,), 1.0, jnp.float32)

    @pl.loop(0, ROWS_PW)
    def _(j):
        for k in range(CH // 16):
            dk = dst_v[j, pl.ds(k * 16, 16)]
            plsc.addupdate_scatter(deg_v, [dk], ones16)

    pltpu.sync_copy(deg_v, deg_hbm.at[pl.ds(g * N, N)])


@functools.cache
def _sc_kernels():
    mesh = plsc.VectorSubcoreMesh(
        core_axis_name="c", subcore_axis_name="s",
        num_cores=NC, num_subcores=NS)
    params = pltpu.CompilerParams(needs_layout_passes=False)
    agg = pl.kernel(
        _sc_agg_body,
        compiler_params=params,
        out_type=jax.ShapeDtypeStruct((NC, NS, NPT, D), jnp.float32),
        mesh=mesh,
        scratch_types=[
            pltpu.VMEM((ROWS_PW, CH), jnp.int32),
            pltpu.VMEM((ROWS_PW, CH), jnp.int32),
            pltpu.VMEM((CH, D), jnp.float32),
            pltpu.SemaphoreType.DMA,
            pltpu.VMEM_SHARED((N, D), jnp.float32),
        ],
    )
    deg = pl.kernel(
        _sc_deg_body,
        compiler_params=params,
        out_type=jax.ShapeDtypeStruct((NW * N,), jnp.float32),
        mesh=mesh,
        scratch_types=[
            pltpu.VMEM((ROWS_PW, CH), jnp.int32),
            pltpu.VMEM((N,), jnp.float32),
        ],
    )
    return agg, deg


BM = 400
_GRID = N // BM


def _tc_layer_body(relu_out, acc_ref, deg_ref, x_ref, wl_ref, bl_ref, wr_ref,
                   *outs):
    deg = jnp.sum(deg_ref[...], axis=0)
    acc = acc_ref[0] + acc_ref[1]
    mean = acc * (1.0 / jnp.maximum(deg, 1.0))
    h1 = (lax.dot_general(mean, wl_ref[...], (((1,), (1,)), ((), ())),
                          preferred_element_type=jnp.float32)
          + bl_ref[...]
          + lax.dot_general(x_ref[...], wr_ref[...], (((1,), (1,)), ((), ())),
                            preferred_element_type=jnp.float32))
    outs[0][...] = h1
    if relu_out:
        outs[1][...] = jnp.maximum(h1, 0.0)


def _make_tc(relu_out):
    n_out = 2 if relu_out else 1
    return pl.pallas_call(
        functools.partial(_tc_layer_body, relu_out),
        grid=(_GRID,),
        in_specs=[
            pl.BlockSpec((NC, BM, D), lambda i: (0, i, 0)),
            pl.BlockSpec((NW, BM, 1), lambda i: (0, i, 0)),
            pl.BlockSpec((BM, D), lambda i: (i, 0)),
            pl.BlockSpec((D, D), lambda i: (0, 0)),
            pl.BlockSpec((1, D), lambda i: (0, 0)),
            pl.BlockSpec((D, D), lambda i: (0, 0)),
        ],
        out_specs=[pl.BlockSpec((BM, D), lambda i: (i, 0))] * n_out,
        out_shape=[jax.ShapeDtypeStruct((N, D), jnp.float32)] * n_out,
    )


_tc_layer_relu = _make_tc(True)
_tc_layer_last = _make_tc(False)


def kernel(x, edge_index, W_l0, b_l0, W_r0, W_l1, b_l1, W_r1):
    sc_agg, sc_deg = _sc_kernels()
    src = edge_index[0].astype(jnp.int32).reshape(NW, ROWS_PW, CH)
    dst = edge_index[1].astype(jnp.int32).reshape(NW, ROWS_PW, CH)
    degp = sc_deg(dst)
    deg = degp.reshape(NW, N, 1)
    acc0 = sc_agg(x, src, dst).reshape(NC, N, D)
    h1, h = _tc_layer_relu(acc0, deg, x, W_l0, b_l0.reshape(1, D), W_r0)
    acc1 = sc_agg(h, src, dst).reshape(NC, N, D)
    (h2,) = _tc_layer_last(acc1, deg, h, W_l1, b_l1.reshape(1, D), W_r1)
    return (h1, h2)
